# baseline (speedup 1.0000x reference)
import functools

import jax
import jax.numpy as jnp
from jax import lax
from jax.experimental import pallas as pl
from jax.experimental.pallas import tpu as pltpu
from jax.experimental.pallas import tpu_sc as plsc

NC = 2
NS = 16
NW = NC * NS

C = 1024
NBUF = 2


def _gather_body(idx_hbm, w_hbm, s_hbm, m_hbm, q_out, s_out, m_out,
                 idx_v, rows_v, s_v, m_v, sem_g, sem_w, *, rpw):
  wid = lax.axis_index("s") * NC + lax.axis_index("c")
  base0 = wid * rpw
  nchunks = rpw // C

  pending_gather = [None] * NBUF
  pending_wb = [None] * NBUF

  def start(ci, sl):
    base = pl.multiple_of(base0 + ci * C, C)
    if pending_wb[sl] is not None:
      for c in pending_wb[sl]:
        c.wait()
      pending_wb[sl] = None
    pltpu.sync_copy(idx_hbm.at[pl.ds(base, C)], idx_v.at[sl])
    pending_gather[sl] = (
        pltpu.async_copy(w_hbm.at[idx_v.at[sl]], rows_v.at[sl],
                         sem_g.at[sl, 0]),
        pltpu.async_copy(s_hbm.at[idx_v.at[sl]], s_v.at[sl], sem_g.at[sl, 1]),
        pltpu.async_copy(m_hbm.at[idx_v.at[sl]], m_v.at[sl], sem_g.at[sl, 2]),
    )

  def drain(ci, sl):
    base = pl.multiple_of(base0 + ci * C, C)
    for c in pending_gather[sl]:
      c.wait()
    pending_gather[sl] = None
    pending_wb[sl] = (
        pltpu.async_copy(rows_v.at[sl], q_out.at[pl.ds(base, C)], sem_w.at[sl, 0]),
        pltpu.async_copy(s_v.at[sl], s_out.at[pl.ds(base, C)], sem_w.at[sl, 1]),
        pltpu.async_copy(m_v.at[sl], m_out.at[pl.ds(base, C)], sem_w.at[sl, 2]),
    )

  start(0, 0)
  for ci in range(nchunks):
    if ci + 1 < nchunks:
      start(ci + 1, (ci + 1) % NBUF)
    drain(ci, ci % NBUF)
  for sl in range(NBUF):
    if pending_wb[sl] is not None:
      for c in pending_wb[sl]:
        c.wait()


def _dequant_body(q_ref, s_ref, m_ref, oe_ref, oo_ref):
  w = q_ref[...]
  s = s_ref[...]
  t = s * m_ref[...]
  b0 = (w << 24) >> 24
  b1 = (w << 16) >> 24
  b2 = (w << 8) >> 24
  b3 = w >> 24
  def bf16_bits(b):
    y = jax.lax.bitcast_convert_type(b.astype(jnp.float32) * s + t, jnp.int32)
    return ((y + 0x7FFF + ((y >> 16) & 1)) >> 16) & 0xFFFF

  r0 = bf16_bits(b0)
  r1 = bf16_bits(b1)
  r2 = bf16_bits(b2)
  r3 = bf16_bits(b3)
  oe_ref[...] = r0 | (r1 << 16)
  oo_ref[...] = r2 | (r3 << 16)


def kernel(idx, weight, scales, means):
  B, L = idx.shape
  V, D = weight.shape
  BL = B * L
  W = D // 4
  rpw = BL // NW

  idxf = idx.reshape(BL)
  w32 = jax.lax.bitcast_convert_type(
      weight.reshape(V, W, 4), jnp.int32)
  sf = scales.reshape(V)
  mf = means.reshape(V)

  mesh = plsc.VectorSubcoreMesh(core_axis_name="c", subcore_axis_name="s")
  q, s_g, m_g = pl.kernel(
      functools.partial(_gather_body, rpw=rpw),
      out_type=[
          jax.ShapeDtypeStruct((BL, W), jnp.int32),
          jax.ShapeDtypeStruct((BL,), jnp.float32),
          jax.ShapeDtypeStruct((BL,), jnp.float32),
      ],
      mesh=mesh,
      compiler_params=pltpu.CompilerParams(
          needs_layout_passes=False, use_tc_tiling_on_sc=False),
      scratch_types=[
          pltpu.VMEM((NBUF, C), jnp.int32),
          pltpu.VMEM((NBUF, C, W), jnp.int32),
          pltpu.VMEM((NBUF, C), jnp.float32),
          pltpu.VMEM((NBUF, C), jnp.float32),
          pltpu.SemaphoreType.DMA((NBUF, 3)),
          pltpu.SemaphoreType.DMA((NBUF, 3)),
      ],
  )(idxf, w32, sf, mf)

  NR = BL * W // 128
  q2 = q.reshape(NR, 128)
  s2 = jnp.repeat(s_g, W).reshape(NR, 128)
  m2 = jnp.repeat(m_g, W).reshape(NR, 128)

  Bq = 256
  oe, oo = pl.pallas_call(
      _dequant_body,
      grid=(NR // Bq,),
      in_specs=[
          pl.BlockSpec((Bq, 128), lambda i: (i, 0)),
          pl.BlockSpec((Bq, 128), lambda i: (i, 0)),
          pl.BlockSpec((Bq, 128), lambda i: (i, 0)),
      ],
      out_specs=[
          pl.BlockSpec((Bq, 128), lambda i: (i, 0)),
          pl.BlockSpec((Bq, 128), lambda i: (i, 0)),
      ],
      out_shape=[
          jax.ShapeDtypeStruct((NR, 128), jnp.int32),
          jax.ShapeDtypeStruct((NR, 128), jnp.int32),
      ],
  )(q2, s2, m2)

  out32 = jnp.stack([oe, oo], axis=-1)
  out = jax.lax.bitcast_convert_type(out32, jnp.bfloat16)
  return out.reshape(B, L, D)


# speedup vs baseline: 10.8426x; 10.8426x over previous
import functools

import jax
import jax.numpy as jnp
from jax import lax
from jax.experimental import pallas as pl
from jax.experimental.pallas import tpu as pltpu
from jax.experimental.pallas import tpu_sc as plsc

NC = 2
NS = 16
NW = NC * NS

C = 1024
NBUF = 2


def _gather_body(idx_hbm, w_hbm, s_hbm, m_hbm, rows_out, s_out, m_out,
                 idx_v, rows_v, s_v, m_v, sem_g, sem_w, *, rpw):
  wid = lax.axis_index("s") * NC + lax.axis_index("c")
  base0 = wid * rpw
  nchunks = rpw // C

  pending_gather = [None] * NBUF
  pending_wb = [None] * NBUF

  def start(ci, sl):
    base = pl.multiple_of(base0 + ci * C, C)
    if pending_wb[sl] is not None:
      for c in pending_wb[sl]:
        c.wait()
      pending_wb[sl] = None
    pltpu.sync_copy(idx_hbm.at[pl.ds(base, C)], idx_v.at[sl])
    pending_gather[sl] = (
        pltpu.async_copy(w_hbm.at[idx_v.at[sl]], rows_v.at[sl], sem_g.at[sl, 0]),
        pltpu.async_copy(s_hbm.at[idx_v.at[sl]], s_v.at[sl], sem_g.at[sl, 1]),
        pltpu.async_copy(m_hbm.at[idx_v.at[sl]], m_v.at[sl], sem_g.at[sl, 2]),
    )

  def drain(ci, sl):
    base = pl.multiple_of(base0 + ci * C, C)
    for c in pending_gather[sl]:
      c.wait()
    pending_gather[sl] = None
    pending_wb[sl] = (
        pltpu.async_copy(rows_v.at[sl], rows_out.at[pl.ds(base, C)], sem_w.at[sl, 0]),
        pltpu.async_copy(s_v.at[sl], s_out.at[pl.ds(base, C)], sem_w.at[sl, 1]),
        pltpu.async_copy(m_v.at[sl], m_out.at[pl.ds(base, C)], sem_w.at[sl, 2]),
    )

  start(0, 0)
  for ci in range(nchunks):
    if ci + 1 < nchunks:
      start(ci + 1, (ci + 1) % NBUF)
    drain(ci, ci % NBUF)
  for sl in range(NBUF):
    if pending_wb[sl] is not None:
      for c in pending_wb[sl]:
        c.wait()


def _dequant_body(q_ref, s_ref, m_ref, o_ref):
  q = q_ref[...].astype(jnp.float32)
  s = s_ref[...]
  m = m_ref[...]
  o_ref[...] = (s * (q + m)).astype(jnp.bfloat16)


def kernel(idx, weight, scales, means):
  B, L = idx.shape
  V, D = weight.shape
  BL = B * L
  rpw = BL // NW

  idxf = idx.reshape(BL)
  wlin = jax.lax.optimization_barrier(weight.reshape(V * D))
  w2 = wlin.reshape(V, D)
  sf = scales.reshape(V)
  mf = means.reshape(V)

  mesh = plsc.VectorSubcoreMesh(core_axis_name="c", subcore_axis_name="s")
  rows, s_g, m_g = pl.kernel(
      functools.partial(_gather_body, rpw=rpw),
      out_type=[
          jax.ShapeDtypeStruct((BL, D), jnp.int8),
          jax.ShapeDtypeStruct((BL,), jnp.float32),
          jax.ShapeDtypeStruct((BL,), jnp.float32),
      ],
      mesh=mesh,
      compiler_params=pltpu.CompilerParams(
          needs_layout_passes=False, use_tc_tiling_on_sc=False),
      scratch_types=[
          pltpu.VMEM((NBUF, C), jnp.int32),
          pltpu.VMEM((NBUF, C, D), jnp.int8),
          pltpu.VMEM((NBUF, C), jnp.float32),
          pltpu.VMEM((NBUF, C), jnp.float32),
          pltpu.SemaphoreType.DMA((NBUF, 3)),
          pltpu.SemaphoreType.DMA((NBUF, 3)),
      ],
  )(idxf, w2, sf, mf)

  Bt = 2048
  out = pl.pallas_call(
      _dequant_body,
      grid=(BL // Bt,),
      in_specs=[
          pl.BlockSpec((Bt, D), lambda i: (i, 0)),
          pl.BlockSpec((Bt, 1), lambda i: (i, 0)),
          pl.BlockSpec((Bt, 1), lambda i: (i, 0)),
      ],
      out_specs=pl.BlockSpec((Bt, D), lambda i: (i, 0)),
      out_shape=jax.ShapeDtypeStruct((BL, D), jnp.bfloat16),
      compiler_params=pltpu.CompilerParams(
          dimension_semantics=("parallel",)),
  )(rows, s_g.reshape(BL, 1), m_g.reshape(BL, 1))
  return out.reshape(B, L, D)


# speedup vs baseline: 11.1432x; 1.0277x over previous
import functools

import jax
import jax.numpy as jnp
from jax import lax
from jax.experimental import pallas as pl
from jax.experimental.pallas import tpu as pltpu
from jax.experimental.pallas import tpu_sc as plsc

NC = 2
NS = 16
NW = NC * NS

C = 1024
NBUF = 2


def _gather_body(idx_hbm, w_hbm, s_hbm, m_hbm, rows_out, s_out, m_out,
                 idx_v, rows_v, s_v, m_v, sem_g, sem_w, *, rpw):
  wid = lax.axis_index("s") * NC + lax.axis_index("c")
  base0 = wid * rpw
  nchunks = rpw // C

  pending_gather = [None] * NBUF
  pending_wb = [None] * NBUF

  def start(ci, sl):
    base = pl.multiple_of(base0 + ci * C, C)
    if pending_wb[sl] is not None:
      for c in pending_wb[sl]:
        c.wait()
      pending_wb[sl] = None
    pltpu.sync_copy(idx_hbm.at[pl.ds(base, C)], idx_v.at[sl])
    pending_gather[sl] = (
        pltpu.async_copy(w_hbm.at[idx_v.at[sl]], rows_v.at[sl], sem_g.at[sl, 0]),
        pltpu.async_copy(s_hbm.at[idx_v.at[sl]], s_v.at[sl], sem_g.at[sl, 1]),
        pltpu.async_copy(m_hbm.at[idx_v.at[sl]], m_v.at[sl], sem_g.at[sl, 2]),
    )

  def drain(ci, sl):
    base = pl.multiple_of(base0 + ci * C, C)
    for c in pending_gather[sl]:
      c.wait()
    pending_gather[sl] = None
    pending_wb[sl] = (
        pltpu.async_copy(rows_v.at[sl], rows_out.at[pl.ds(base, C)], sem_w.at[sl, 0]),
        pltpu.async_copy(s_v.at[sl], s_out.at[pl.ds(base, C)], sem_w.at[sl, 1]),
        pltpu.async_copy(m_v.at[sl], m_out.at[pl.ds(base, C)], sem_w.at[sl, 2]),
    )

  start(0, 0)
  for ci in range(nchunks):
    if ci + 1 < nchunks:
      start(ci + 1, (ci + 1) % NBUF)
    drain(ci, ci % NBUF)
  for sl in range(NBUF):
    if pending_wb[sl] is not None:
      for c in pending_wb[sl]:
        c.wait()


def _dequant_body(q_ref, s_ref, m_ref, o_ref):
  q = q_ref[...].astype(jnp.float32)
  s = s_ref[...]
  m = m_ref[...]
  o_ref[...] = (s * (q + m)).astype(jnp.bfloat16)


def kernel(idx, weight, scales, means):
  B, L = idx.shape
  V, D = weight.shape
  BL = B * L
  rpw = BL // NW

  idxf = idx.reshape(BL)
  wlin = jax.lax.optimization_barrier(weight.reshape(V * D))
  w2 = wlin.reshape(V, D)
  sf = scales.reshape(V)
  mf = means.reshape(V)

  mesh = plsc.VectorSubcoreMesh(core_axis_name="c", subcore_axis_name="s")
  rows, s_g, m_g = pl.kernel(
      functools.partial(_gather_body, rpw=rpw),
      out_type=[
          jax.ShapeDtypeStruct((BL, D), jnp.int8),
          jax.ShapeDtypeStruct((BL,), jnp.float32),
          jax.ShapeDtypeStruct((BL,), jnp.float32),
      ],
      mesh=mesh,
      compiler_params=pltpu.CompilerParams(
          needs_layout_passes=False, use_tc_tiling_on_sc=False),
      scratch_types=[
          pltpu.VMEM((NBUF, C), jnp.int32),
          pltpu.VMEM((NBUF, C, D), jnp.int8),
          pltpu.VMEM((NBUF, C), jnp.float32),
          pltpu.VMEM((NBUF, C), jnp.float32),
          pltpu.SemaphoreType.DMA((NBUF, 3)),
          pltpu.SemaphoreType.DMA((NBUF, 3)),
      ],
  )(idxf, w2, sf, mf)

  Bt = 4096
  out = pl.pallas_call(
      _dequant_body,
      grid=(BL // Bt,),
      in_specs=[
          pl.BlockSpec((Bt, D), lambda i: (i, 0)),
          pl.BlockSpec((Bt, 1), lambda i: (i, 0)),
          pl.BlockSpec((Bt, 1), lambda i: (i, 0)),
      ],
      out_specs=pl.BlockSpec((Bt, D), lambda i: (i, 0)),
      out_shape=jax.ShapeDtypeStruct((BL, D), jnp.bfloat16),
  )(rows, s_g.reshape(BL, 1), m_g.reshape(BL, 1))
  return out.reshape(B, L, D)
